# baseline (speedup 1.0000x reference)
import functools

import jax
import jax.numpy as jnp
from jax import lax
from jax.experimental import pallas as pl
from jax.experimental.pallas import tpu as pltpu
from jax.experimental.pallas import tpu_sc as plsc

N = 10000
D = 128
HD = D // 2
E = 320000
NP = 10240
NC = 2
NS = 16
NW = NC * NS
K = 80
ECH = E // K
CPW = ECH // NW
CPT = ECH // NS
RPT = NP // NS
NB = 5
NT = CPW // NB

_MESH = plsc.VectorSubcoreMesh(core_axis_name="c", subcore_axis_name="s")

_GATHER_DNUMS = lax.GatherDimensionNumbers(
    offset_dims=(), collapsed_slice_dims=(0,), start_index_map=(0,))


def _lane_broadcast(vec, lane):
    idx = jnp.full((16, 1), lane, jnp.int32)
    return lax.gather(vec, idx, _GATHER_DNUMS, slice_sizes=(1,),
                      mode=lax.GatherScatterMode.PROMISE_IN_BOUNDS)


def _zero_1d(ref, n):
    def body(i, _):
        ref[pl.ds(i * 16, 16)] = jnp.zeros((16,), jnp.float32)
        return 0
    lax.fori_loop(0, n // 16, body, 0, unroll=8)


XB = 80
NXB = RPT // XB


@functools.partial(
    pl.kernel,
    mesh=_MESH,
    out_type=[jax.ShapeDtypeStruct((NP,), jnp.float32)]
    + [jax.ShapeDtypeStruct((NP, HD), jnp.float32)] * 2,
    scratch_types=[
        pltpu.VMEM((CPT, K), jnp.int32),
        pltpu.VMEM((CPT, K), jnp.float32),
        pltpu.VMEM((RPT,), jnp.float32),
        pltpu.VMEM((XB, D), jnp.float32),
        pltpu.VMEM((XB, D), jnp.float32),
        pltpu.VMEM((XB, HD), jnp.float32),
        pltpu.VMEM((XB, HD), jnp.float32),
        pltpu.VMEM_SHARED((NP,), jnp.float32),
        pltpu.SemaphoreType.DMA,
        pltpu.SemaphoreType.DMA,
    ],
    compiler_params=pltpu.CompilerParams(use_tc_tiling_on_sc=False),
)
def _sc_degree(dst_hbm, w_hbm, x_hbm, dis_hbm, xlo_hbm, xhi_hbm,
               dst_st, w_st, buf, xa, xb, slo, shi, deg_sh, semx, semd):
    c = lax.axis_index("c")
    s = lax.axis_index("s")

    @pl.when(c == 0)
    def _():
        _zero_1d(buf, RPT)
        pltpu.sync_copy(buf, deg_sh.at[pl.ds(s * RPT, RPT)])
        pltpu.sync_copy(dst_hbm.at[s], dst_st)
        pltpu.sync_copy(w_hbm.at[s], w_st)
        plsc.subcore_barrier()

        def fire(ch, _):
            pltpu.async_copy(w_st.at[ch], deg_sh.at[dst_st.at[ch]], semd,
                             add=True)
            return 0
        lax.fori_loop(0, CPT, fire, 0)
        pltpu.async_copy(x_hbm.at[pl.ds(s * RPT, XB)], xa, semx)

        def drain(ch, _):
            pltpu.make_async_copy(w_st.at[ch], deg_sh.at[dst_st.at[ch]],
                                  semd).wait()
            return 0
        lax.fori_loop(0, CPT, drain, 0)

        plsc.subcore_barrier()
        pltpu.sync_copy(deg_sh.at[pl.ds(s * RPT, RPT)], buf)

        def newton(i, _):
            x = buf[pl.ds(i * 16, 16)] + 1.0
            y = jnp.where(x < 4.0, 0.5,
                jnp.where(x < 64.0, 0.125,
                jnp.where(x < 1024.0, 0.03125,
                jnp.where(x < 16384.0, 7.8125e-3,
                jnp.where(x < 262144.0, 1.953125e-3, 4.8828125e-4)))))
            for _ in range(8):
                y = y * (1.5 - 0.5 * x * y * y)
            buf[pl.ds(i * 16, 16)] = y
            return 0
        lax.fori_loop(0, RPT // 16, newton, 0)
        pltpu.sync_copy(buf, dis_hbm.at[pl.ds(s * RPT, RPT)])

        xbufs = (xa, xb)

        def xblk(kblk, _):
            for p in range(2):
                blk = kblk * 2 + p
                xin = xbufs[p]
                base = s * RPT + blk * XB
                pltpu.make_async_copy(x_hbm.at[pl.ds(base, XB)], xin,
                                     semx).wait()

                @pl.when(blk + 1 < NXB)
                def _(p=p, blk=blk):
                    pltpu.async_copy(
                        x_hbm.at[pl.ds(s * RPT + (blk + 1) * XB, XB)],
                        xbufs[1 - p], semx)

                def srow(g, _, xin=xin, blk=blk):
                    dv = buf[pl.ds(blk * XB + g * 16, 16)]
                    for l in range(16):
                        db = _lane_broadcast(dv, l)
                        r = g * 16 + l
                        for j in range(D // 16):
                            tgt = slo if j < HD // 16 else shi
                            tj = j if j < HD // 16 else j - HD // 16
                            tgt[r, pl.ds(tj * 16, 16)] = (
                                xin[r, pl.ds(j * 16, 16)] * db)
                    return 0
                lax.fori_loop(0, XB // 16, srow, 0)

                pltpu.sync_copy(slo, xlo_hbm.at[pl.ds(base, XB)])
                pltpu.sync_copy(shi, xhi_hbm.at[pl.ds(base, XB)])
            return 0
        lax.fori_loop(0, NXB // 2, xblk, 0)


@functools.partial(
    pl.kernel,
    mesh=_MESH,
    out_type=[jax.ShapeDtypeStruct((NC, NP, HD), jnp.float32)] * 2,
    scratch_types=[
        pltpu.VMEM((CPW, K), jnp.int32),
        pltpu.VMEM((CPW, K), jnp.int32),
        pltpu.VMEM((CPW, K), jnp.float32),
    ] + [pltpu.VMEM((K, HD), jnp.float32)] * (2 * NB)
      + [pltpu.VMEM_SHARED((NP, HD), jnp.float32)]
      + [pltpu.SemaphoreType.DMA] * (2 * NB),
    compiler_params=pltpu.CompilerParams(use_tc_tiling_on_sc=False),
)
def _sc_aggregate(src_hbm, dst_hbm, w_hbm, xlo_hbm, xhi_hbm,
                  outlo_hbm, outhi_hbm,
                  src_st, dst_st, w_st,
                  rows0, rows1, rows2, rows3, rows4,
                  scl0, scl1, scl2, scl3, scl4, s_sh,
                  semg0, semg1, semg2, semg3, semg4,
                  sems0, sems1, sems2, sems3, sems4):
    c = lax.axis_index("c")
    s = lax.axis_index("s")
    wid = c * NS + s
    rows = (rows0, rows1, rows2, rows3, rows4)
    scl = (scl0, scl1, scl2, scl3, scl4)
    semg = (semg0, semg1, semg2, semg3, semg4)
    sems = (sems0, sems1, sems2, sems3, sems4)

    pltpu.sync_copy(src_hbm.at[wid], src_st)
    pltpu.sync_copy(dst_hbm.at[wid], dst_st)
    pltpu.sync_copy(w_hbm.at[wid], w_st)

    def zero_accum():
        def zrows(i, _):
            r = i // (HD // 16)
            j = i % (HD // 16)
            rows0[r, pl.ds(j * 16, 16)] = jnp.zeros((16,), jnp.float32)
            return 0
        lax.fori_loop(0, K * (HD // 16), zrows, 0, unroll=8)
        for i in range(RPT // K):
            pltpu.async_copy(rows0, s_sh.at[pl.ds(s * RPT + i * K, K)], semg0)
        for i in range(RPT // K):
            pltpu.make_async_copy(rows0, s_sh.at[pl.ds(s * RPT, K)],
                                  semg0).wait()

    GA = 2

    def run_pass(x_hbm, out_hbm):
        zero_accum()
        plsc.subcore_barrier()

        for b in range(GA):
            pltpu.async_copy(x_hbm.at[src_st.at[b]], rows[b], semg[b])

        def outer(t, _):
            for b in range(NB):
                ch = t * NB + b
                rb = rows[b]
                sb = scl[b]
                bn = (b + GA) % NB

                @pl.when(ch - NB >= 0)
                def _(sb=sb, b=b):
                    pltpu.make_async_copy(sb, s_sh.at[dst_st.at[0]],
                                          sems[b]).wait()

                @pl.when(ch + GA < CPW)
                def _(bn=bn, ch=ch):
                    pltpu.async_copy(x_hbm.at[src_st.at[ch + GA]], rows[bn],
                                     semg[bn])

                pltpu.make_async_copy(x_hbm.at[src_st.at[ch]], rb,
                                      semg[b]).wait()

                def grp(g, _, rb=rb, sb=sb, ch=ch):
                    wv = w_st[ch, pl.ds(g * 16, 16)]
                    for l in range(16):
                        wb = _lane_broadcast(wv, l)
                        e = g * 16 + l
                        for j in range(HD // 16):
                            sb[e, pl.ds(j * 16, 16)] = (
                                rb[e, pl.ds(j * 16, 16)] * wb)
                    return 0
                lax.fori_loop(0, K // 16, grp, 0)

                pltpu.async_copy(sb, s_sh.at[dst_st.at[ch]], sems[b],
                                 add=True)
            return 0
        lax.fori_loop(0, NT, outer, 0)

        for b in range(NB):
            pltpu.make_async_copy(scl[b], s_sh.at[dst_st.at[0]],
                                  sems[b]).wait()

        plsc.subcore_barrier()
        pltpu.sync_copy(s_sh.at[pl.ds(s * RPT, RPT)],
                        out_hbm.at[c, pl.ds(s * RPT, RPT)])
        plsc.subcore_barrier()

    run_pass(xlo_hbm, outlo_hbm)
    run_pass(xhi_hbm, outhi_hbm)


def _tc_gates_body(slo_ref, shi_ref, xlo_ref, xhi_ref, dis_ref,
                   wci, bci, wli, bli,
                   wcg, bcg, wlg, blg,
                   wco, bco, wlo, blo,
                   o_ref, hn_ref, cn_ref):
    dis = dis_ref[...]
    ylo = (slo_ref[0] + slo_ref[1] + xlo_ref[...]) * dis[:, None]
    yhi = (shi_ref[0] + shi_ref[1] + xhi_ref[...]) * dis[:, None]

    def z(wc, bc, wl, bl):
        wl_top = wl[:D, :]
        weff = jnp.dot(wc[...], wl_top, preferred_element_type=jnp.float32)
        beff = jnp.dot(bc[...], wl_top, preferred_element_type=jnp.float32) + bl[...]
        return (jnp.dot(ylo, weff[:HD, :], preferred_element_type=jnp.float32)
                + jnp.dot(yhi, weff[HD:, :], preferred_element_type=jnp.float32)
                + beff)

    gi = jax.nn.sigmoid(z(wci, bci, wli, bli))
    gg = jnp.tanh(z(wcg, bcg, wlg, blg))
    go = jax.nn.sigmoid(z(wco, bco, wlo, blo))
    cn = gi * gg
    o_ref[...] = go
    cn_ref[...] = cn
    hn_ref[...] = go * jnp.tanh(cn)


_ROWB = 2048
_GRID = NP // _ROWB


def _row_spec(cols=D):
    return pl.BlockSpec((_ROWB, cols), lambda i: (i, 0))


def _full_spec(shape):
    return pl.BlockSpec(shape, lambda i: (0,) * len(shape))


_tc_gates = pl.pallas_call(
    _tc_gates_body,
    grid=(_GRID,),
    in_specs=[
        pl.BlockSpec((NC, _ROWB, HD), lambda i: (0, i, 0)),
        pl.BlockSpec((NC, _ROWB, HD), lambda i: (0, i, 0)),
        _row_spec(HD),
        _row_spec(HD),
        pl.BlockSpec((_ROWB,), lambda i: (i,)),
    ] + [_full_spec((D, D)), _full_spec((1, D)),
         _full_spec((2 * D, D)), _full_spec((1, D))] * 3,
    out_specs=[_row_spec()] * 3,
    out_shape=[jax.ShapeDtypeStruct((NP, D), jnp.float32)] * 3,
)


def kernel(X, edge_index, edge_weight,
           Wc_i, bc_i, Wl_i, bl_i,
           Wc_f, bc_f, Wl_f, bl_f,
           Wc_g, bc_g, Wl_g, bl_g,
           Wc_o, bc_o, Wl_o, bl_o):
    src3 = edge_index[0].reshape(NW, CPW, K)
    dst3 = edge_index[1].reshape(NW, CPW, K)
    w3 = edge_weight.reshape(NW, CPW, K)
    dst3d = edge_index[1].reshape(NS, CPT, K)
    w3d = edge_weight.reshape(NS, CPT, K)
    xp = jnp.pad(X, ((0, NP - N), (0, 0)))

    dis, xlo, xhi = _sc_degree(dst3d, w3d, xp)
    slo, shi = _sc_aggregate(src3, dst3, w3, xlo, xhi)

    r1 = lambda b: b.reshape(1, D)
    o, hn, cn = _tc_gates(
        slo, shi, xlo, xhi, dis,
        Wc_i, r1(bc_i), Wl_i, r1(bl_i),
        Wc_g, r1(bc_g), Wl_g, r1(bl_g),
        Wc_o, r1(bc_o), Wl_o, r1(bl_o),
    )
    return (o[:N], hn[:N], cn[:N])


# speedup vs baseline: 1.0460x; 1.0460x over previous
import functools

import jax
import jax.numpy as jnp
from jax import lax
from jax.experimental import pallas as pl
from jax.experimental.pallas import tpu as pltpu
from jax.experimental.pallas import tpu_sc as plsc

N = 10000
D = 128
HD = D // 2
E = 320000
NP = 10240
NC = 2
NS = 16
NW = NC * NS
K = 80
ECH = E // K
CPW = ECH // NW
CPT = ECH // NS
RPT = NP // NS
NB = 5
NT = CPW // NB

_MESH = plsc.VectorSubcoreMesh(core_axis_name="c", subcore_axis_name="s")

_GATHER_DNUMS = lax.GatherDimensionNumbers(
    offset_dims=(), collapsed_slice_dims=(0,), start_index_map=(0,))


def _lane_broadcast(vec, lane):
    idx = jnp.full((16, 1), lane, jnp.int32)
    return lax.gather(vec, idx, _GATHER_DNUMS, slice_sizes=(1,),
                      mode=lax.GatherScatterMode.PROMISE_IN_BOUNDS)


def _zero_1d(ref, n):
    def body(i, _):
        ref[pl.ds(i * 16, 16)] = jnp.zeros((16,), jnp.float32)
        return 0
    lax.fori_loop(0, n // 16, body, 0, unroll=8)


XB = 80
NXB = RPT // XB


@functools.partial(
    pl.kernel,
    mesh=_MESH,
    out_type=[jax.ShapeDtypeStruct((NP,), jnp.float32)]
    + [jax.ShapeDtypeStruct((NP, HD), jnp.float32)] * 2,
    scratch_types=[
        pltpu.VMEM((CPT, K), jnp.int32),
        pltpu.VMEM((CPT, K), jnp.float32),
        pltpu.VMEM((RPT,), jnp.float32),
        pltpu.VMEM((XB, D), jnp.float32),
        pltpu.VMEM((XB, D), jnp.float32),
        pltpu.VMEM((XB, HD), jnp.float32),
        pltpu.VMEM((XB, HD), jnp.float32),
        pltpu.VMEM_SHARED((NP,), jnp.float32),
        pltpu.SemaphoreType.DMA,
        pltpu.SemaphoreType.DMA,
    ],
    compiler_params=pltpu.CompilerParams(use_tc_tiling_on_sc=False),
)
def _sc_degree(dst_hbm, w_hbm, x_hbm, dis_hbm, xlo_hbm, xhi_hbm,
               dst_st, w_st, buf, xa, xb, slo, shi, deg_sh, semx, semd):
    c = lax.axis_index("c")
    s = lax.axis_index("s")

    @pl.when(c == 0)
    def _():
        _zero_1d(buf, RPT)
        pltpu.sync_copy(buf, deg_sh.at[pl.ds(s * RPT, RPT)])
        pltpu.sync_copy(dst_hbm.at[s], dst_st)
        pltpu.sync_copy(w_hbm.at[s], w_st)
        plsc.subcore_barrier()

        def fire(ch, _):
            pltpu.async_copy(w_st.at[ch], deg_sh.at[dst_st.at[ch]], semd,
                             add=True)
            return 0
        lax.fori_loop(0, CPT, fire, 0)
        pltpu.async_copy(x_hbm.at[pl.ds(s * RPT, XB)], xa, semx)

        def drain(ch, _):
            pltpu.make_async_copy(w_st.at[ch], deg_sh.at[dst_st.at[ch]],
                                  semd).wait()
            return 0
        lax.fori_loop(0, CPT, drain, 0)

        plsc.subcore_barrier()
        pltpu.sync_copy(deg_sh.at[pl.ds(s * RPT, RPT)], buf)

        def newton(i, _):
            x = buf[pl.ds(i * 16, 16)] + 1.0
            y = jnp.where(x < 4.0, 0.5,
                jnp.where(x < 64.0, 0.125,
                jnp.where(x < 1024.0, 0.03125,
                jnp.where(x < 16384.0, 7.8125e-3,
                jnp.where(x < 262144.0, 1.953125e-3, 4.8828125e-4)))))
            for _ in range(8):
                y = y * (1.5 - 0.5 * x * y * y)
            buf[pl.ds(i * 16, 16)] = y
            return 0
        lax.fori_loop(0, RPT // 16, newton, 0)
        pltpu.sync_copy(buf, dis_hbm.at[pl.ds(s * RPT, RPT)])

        xbufs = (xa, xb)

        def xblk(kblk, _):
            for p in range(2):
                blk = kblk * 2 + p
                xin = xbufs[p]
                base = s * RPT + blk * XB
                pltpu.make_async_copy(x_hbm.at[pl.ds(base, XB)], xin,
                                     semx).wait()

                @pl.when(blk + 1 < NXB)
                def _(p=p, blk=blk):
                    pltpu.async_copy(
                        x_hbm.at[pl.ds(s * RPT + (blk + 1) * XB, XB)],
                        xbufs[1 - p], semx)

                def srow(g, _, xin=xin, blk=blk):
                    dv = buf[pl.ds(blk * XB + g * 16, 16)]
                    for l in range(16):
                        db = _lane_broadcast(dv, l)
                        r = g * 16 + l
                        for j in range(D // 16):
                            tgt = slo if j < HD // 16 else shi
                            tj = j if j < HD // 16 else j - HD // 16
                            tgt[r, pl.ds(tj * 16, 16)] = (
                                xin[r, pl.ds(j * 16, 16)] * db)
                    return 0
                lax.fori_loop(0, XB // 16, srow, 0)

                pltpu.sync_copy(slo, xlo_hbm.at[pl.ds(base, XB)])
                pltpu.sync_copy(shi, xhi_hbm.at[pl.ds(base, XB)])
            return 0
        lax.fori_loop(0, NXB // 2, xblk, 0)


@functools.partial(
    pl.kernel,
    mesh=_MESH,
    out_type=[jax.ShapeDtypeStruct((NC, NP, HD), jnp.float32)] * 2,
    scratch_types=[
        pltpu.VMEM((CPW, K), jnp.int32),
        pltpu.VMEM((CPW, K), jnp.int32),
        pltpu.VMEM((CPW, K), jnp.float32),
    ] + [pltpu.VMEM((K, HD), jnp.float32)] * (2 * NB)
      + [pltpu.VMEM_SHARED((NP, HD), jnp.float32)]
      + [pltpu.SemaphoreType.DMA] * (2 * NB),
    compiler_params=pltpu.CompilerParams(use_tc_tiling_on_sc=False),
)
def _sc_aggregate(src_hbm, dst_hbm, w_hbm, xlo_hbm, xhi_hbm,
                  outlo_hbm, outhi_hbm,
                  src_st, dst_st, w_st,
                  rows0, rows1, rows2, rows3, rows4,
                  scl0, scl1, scl2, scl3, scl4, s_sh,
                  semg0, semg1, semg2, semg3, semg4,
                  sems0, sems1, sems2, sems3, sems4):
    c = lax.axis_index("c")
    s = lax.axis_index("s")
    wid = c * NS + s
    rows = (rows0, rows1, rows2, rows3, rows4)
    scl = (scl0, scl1, scl2, scl3, scl4)
    semg = (semg0, semg1, semg2, semg3, semg4)
    sems = (sems0, sems1, sems2, sems3, sems4)

    pltpu.sync_copy(src_hbm.at[wid], src_st)
    pltpu.sync_copy(dst_hbm.at[wid], dst_st)
    pltpu.sync_copy(w_hbm.at[wid], w_st)

    def zero_accum():
        def zrows(i, _):
            r = i // (HD // 16)
            j = i % (HD // 16)
            rows0[r, pl.ds(j * 16, 16)] = jnp.zeros((16,), jnp.float32)
            return 0
        lax.fori_loop(0, K * (HD // 16), zrows, 0, unroll=8)
        for i in range(RPT // K):
            pltpu.async_copy(rows0, s_sh.at[pl.ds(s * RPT + i * K, K)], semg0)
        for i in range(RPT // K):
            pltpu.make_async_copy(rows0, s_sh.at[pl.ds(s * RPT, K)],
                                  semg0).wait()

    GA = 2

    def run_pass(x_hbm, out_hbm):
        zero_accum()
        plsc.subcore_barrier()

        for b in range(GA):
            pltpu.async_copy(x_hbm.at[src_st.at[b]], rows[b], semg[b])

        def outer(t, _):
            for b in range(NB):
                ch = t * NB + b
                rb = rows[b]
                sb = scl[b]
                bn = (b + GA) % NB

                @pl.when(ch - NB >= 0)
                def _(sb=sb, b=b):
                    pltpu.make_async_copy(sb, s_sh.at[dst_st.at[0]],
                                          sems[b]).wait()

                @pl.when(ch + GA < CPW)
                def _(bn=bn, ch=ch):
                    pltpu.async_copy(x_hbm.at[src_st.at[ch + GA]], rows[bn],
                                     semg[bn])

                pltpu.make_async_copy(x_hbm.at[src_st.at[ch]], rb,
                                      semg[b]).wait()

                def grp(g, _, rb=rb, sb=sb, ch=ch):
                    wv = w_st[ch, pl.ds(g * 16, 16)]
                    for l in range(16):
                        wb = _lane_broadcast(wv, l)
                        e = g * 16 + l
                        for j in range(HD // 16):
                            sb[e, pl.ds(j * 16, 16)] = (
                                rb[e, pl.ds(j * 16, 16)] * wb)
                    return 0
                lax.fori_loop(0, K // 16, grp, 0)

                pltpu.async_copy(sb, s_sh.at[dst_st.at[ch]], sems[b],
                                 add=True)
            return 0
        lax.fori_loop(0, NT, outer, 0)

        for b in range(NB):
            pltpu.make_async_copy(scl[b], s_sh.at[dst_st.at[0]],
                                  sems[b]).wait()

        plsc.subcore_barrier()
        pltpu.sync_copy(s_sh.at[pl.ds(s * RPT, RPT)],
                        out_hbm.at[c, pl.ds(s * RPT, RPT)])
        plsc.subcore_barrier()

    run_pass(xlo_hbm, outlo_hbm)
    run_pass(xhi_hbm, outhi_hbm)


def _tc_gates_body(slo_ref, shi_ref, xlo_ref, xhi_ref, dis_ref,
                   wci, bci, wli, bli,
                   wcg, bcg, wlg, blg,
                   wco, bco, wlo, blo,
                   o_ref, hn_ref, cn_ref):
    dis = dis_ref[...]
    ylo = (slo_ref[0] + slo_ref[1] + xlo_ref[...]) * dis[:, None]
    yhi = (shi_ref[0] + shi_ref[1] + xhi_ref[...]) * dis[:, None]

    def z(wc, bc, wl, bl):
        wl_top = wl[:D, :]
        weff = jnp.dot(wc[...], wl_top, preferred_element_type=jnp.float32)
        beff = jnp.dot(bc[...], wl_top, preferred_element_type=jnp.float32) + bl[...]
        return (jnp.dot(ylo, weff[:HD, :], preferred_element_type=jnp.float32)
                + jnp.dot(yhi, weff[HD:, :], preferred_element_type=jnp.float32)
                + beff)

    gi = jax.nn.sigmoid(z(wci, bci, wli, bli))
    gg = jnp.tanh(z(wcg, bcg, wlg, blg))
    go = jax.nn.sigmoid(z(wco, bco, wlo, blo))
    cn = gi * gg
    o_ref[...] = go
    cn_ref[...] = cn
    hn_ref[...] = go * jnp.tanh(cn)


_ROWB = 2048
_GRID = NP // _ROWB


def _row_spec(cols=D):
    return pl.BlockSpec((_ROWB, cols), lambda i: (i, 0))


def _full_spec(shape):
    return pl.BlockSpec(shape, lambda i: (0,) * len(shape))


_tc_gates = pl.pallas_call(
    _tc_gates_body,
    grid=(_GRID,),
    in_specs=[
        pl.BlockSpec((NC, _ROWB, HD), lambda i: (0, i, 0)),
        pl.BlockSpec((NC, _ROWB, HD), lambda i: (0, i, 0)),
        _row_spec(HD),
        _row_spec(HD),
        pl.BlockSpec((_ROWB,), lambda i: (i,)),
    ] + [_full_spec((D, D)), _full_spec((1, D)),
         _full_spec((2 * D, D)), _full_spec((1, D))] * 3,
    out_specs=[_row_spec()] * 3,
    out_shape=[jax.ShapeDtypeStruct((N, D), jnp.float32)] * 3,
)


def kernel(X, edge_index, edge_weight,
           Wc_i, bc_i, Wl_i, bl_i,
           Wc_f, bc_f, Wl_f, bl_f,
           Wc_g, bc_g, Wl_g, bl_g,
           Wc_o, bc_o, Wl_o, bl_o):
    src3 = edge_index[0].reshape(NW, CPW, K)
    dst3 = edge_index[1].reshape(NW, CPW, K)
    w3 = edge_weight.reshape(NW, CPW, K)
    dst3d = edge_index[1].reshape(NS, CPT, K)
    w3d = edge_weight.reshape(NS, CPT, K)
    xp = jnp.pad(X, ((0, NP - N), (0, 0)))

    dis, xlo, xhi = _sc_degree(dst3d, w3d, xp)
    slo, shi = _sc_aggregate(src3, dst3, w3, xlo, xhi)

    r1 = lambda b: b.reshape(1, D)
    o, hn, cn = _tc_gates(
        slo, shi, xlo, xhi, dis,
        Wc_i, r1(bc_i), Wl_i, r1(bl_i),
        Wc_g, r1(bc_g), Wl_g, r1(bl_g),
        Wc_o, r1(bc_o), Wl_o, r1(bl_o),
    )
    return (o, hn, cn)


# speedup vs baseline: 1.1094x; 1.0607x over previous
import functools

import jax
import jax.numpy as jnp
from jax import lax
from jax.experimental import pallas as pl
from jax.experimental.pallas import tpu as pltpu
from jax.experimental.pallas import tpu_sc as plsc

N = 10000
D = 128
HD = D // 2
E = 320000
NP = 10240
NC = 2
NS = 16
NW = NC * NS
K = 80
ECH = E // K
CPW = ECH // NW
CPT = ECH // NS
RPT = NP // NS
NB = 5
NT = CPW // NB

_MESH = plsc.VectorSubcoreMesh(core_axis_name="c", subcore_axis_name="s")

_GATHER_DNUMS = lax.GatherDimensionNumbers(
    offset_dims=(), collapsed_slice_dims=(0,), start_index_map=(0,))


def _lane_broadcast(vec, lane):
    idx = jnp.full((16, 1), lane, jnp.int32)
    return lax.gather(vec, idx, _GATHER_DNUMS, slice_sizes=(1,),
                      mode=lax.GatherScatterMode.PROMISE_IN_BOUNDS)


def _zero_1d(ref, n):
    def body(i, _):
        ref[pl.ds(i * 16, 16)] = jnp.zeros((16,), jnp.float32)
        return 0
    lax.fori_loop(0, n // 16, body, 0, unroll=8)


XB = 80
NXB = RPT // XB
HRPT = RPT // 2
NXBH = NXB // 2


@functools.partial(
    pl.kernel,
    mesh=_MESH,
    out_type=[jax.ShapeDtypeStruct((NP,), jnp.float32)]
    + [jax.ShapeDtypeStruct((NP, HD), jnp.float32)] * 2,
    scratch_types=[
        pltpu.VMEM((CPT, K), jnp.int32),
        pltpu.VMEM((CPT, K), jnp.float32),
        pltpu.VMEM((RPT,), jnp.float32),
        pltpu.VMEM((XB, D), jnp.float32),
        pltpu.VMEM((XB, D), jnp.float32),
        pltpu.VMEM((XB, HD), jnp.float32),
        pltpu.VMEM((XB, HD), jnp.float32),
        pltpu.VMEM_SHARED((NP,), jnp.float32),
        pltpu.SemaphoreType.DMA,
        pltpu.SemaphoreType.DMA,
    ],
    compiler_params=pltpu.CompilerParams(use_tc_tiling_on_sc=False),
)
def _sc_degree(dst_hbm, w_hbm, x_hbm, dis_hbm, xlo_hbm, xhi_hbm,
               dst_st, w_st, buf, xa, xb, slo, shi, deg_sh, semx, semd):
    c = lax.axis_index("c")
    s = lax.axis_index("s")

    _zero_1d(buf, RPT)
    pltpu.sync_copy(buf, deg_sh.at[pl.ds(s * RPT, RPT)])
    pltpu.sync_copy(dst_hbm.at[s], dst_st)
    pltpu.sync_copy(w_hbm.at[s], w_st)
    plsc.subcore_barrier()

    def fire(ch, _):
        pltpu.async_copy(w_st.at[ch], deg_sh.at[dst_st.at[ch]], semd,
                         add=True)
        return 0
    lax.fori_loop(0, CPT, fire, 0)
    pltpu.async_copy(x_hbm.at[pl.ds(s * RPT + c * HRPT, XB)], xa, semx)

    def drain(ch, _):
        pltpu.make_async_copy(w_st.at[ch], deg_sh.at[dst_st.at[ch]],
                              semd).wait()
        return 0
    lax.fori_loop(0, CPT, drain, 0)

    plsc.subcore_barrier()
    pltpu.sync_copy(deg_sh.at[pl.ds(s * RPT, RPT)], buf)

    def newton(i, _):
        x = buf[pl.ds(i * 16, 16)] + 1.0
        y = jnp.where(x < 4.0, 0.5,
            jnp.where(x < 64.0, 0.125,
            jnp.where(x < 1024.0, 0.03125,
            jnp.where(x < 16384.0, 7.8125e-3,
            jnp.where(x < 262144.0, 1.953125e-3, 4.8828125e-4)))))
        for _ in range(8):
            y = y * (1.5 - 0.5 * x * y * y)
        buf[pl.ds(i * 16, 16)] = y
        return 0
    lax.fori_loop(0, RPT // 16, newton, 0)

    @pl.when(c == 0)
    def _():
        pltpu.sync_copy(buf, dis_hbm.at[pl.ds(s * RPT, RPT)])

    xbufs = (xa, xb)

    def xblk(kblk, _):
        for p in range(2):
            blk = kblk * 2 + p
            xin = xbufs[p]
            off = c * HRPT + blk * XB
            base = s * RPT + off
            pltpu.make_async_copy(x_hbm.at[pl.ds(base, XB)], xin,
                                  semx).wait()

            @pl.when(blk + 1 < NXBH)
            def _(p=p, blk=blk):
                pltpu.async_copy(
                    x_hbm.at[pl.ds(s * RPT + c * HRPT + (blk + 1) * XB, XB)],
                    xbufs[1 - p], semx)

            def srow(g, _, xin=xin, off=off):
                dv = buf[pl.ds(off + g * 16, 16)]
                for l in range(16):
                    db = _lane_broadcast(dv, l)
                    r = g * 16 + l
                    for j in range(D // 16):
                        tgt = slo if j < HD // 16 else shi
                        tj = j if j < HD // 16 else j - HD // 16
                        tgt[r, pl.ds(tj * 16, 16)] = (
                            xin[r, pl.ds(j * 16, 16)] * db)
                return 0
            lax.fori_loop(0, XB // 16, srow, 0)

            pltpu.sync_copy(slo, xlo_hbm.at[pl.ds(base, XB)])
            pltpu.sync_copy(shi, xhi_hbm.at[pl.ds(base, XB)])
        return 0
    lax.fori_loop(0, NXBH // 2, xblk, 0)


@functools.partial(
    pl.kernel,
    mesh=_MESH,
    out_type=[jax.ShapeDtypeStruct((NC, NP, HD), jnp.float32)] * 2,
    scratch_types=[
        pltpu.VMEM((CPW, K), jnp.int32),
        pltpu.VMEM((CPW, K), jnp.int32),
        pltpu.VMEM((CPW, K), jnp.float32),
    ] + [pltpu.VMEM((K, HD), jnp.float32)] * (2 * NB)
      + [pltpu.VMEM_SHARED((NP, HD), jnp.float32)]
      + [pltpu.SemaphoreType.DMA] * (2 * NB),
    compiler_params=pltpu.CompilerParams(use_tc_tiling_on_sc=False),
)
def _sc_aggregate(src_hbm, dst_hbm, w_hbm, xlo_hbm, xhi_hbm,
                  outlo_hbm, outhi_hbm,
                  src_st, dst_st, w_st,
                  rows0, rows1, rows2, rows3, rows4,
                  scl0, scl1, scl2, scl3, scl4, s_sh,
                  semg0, semg1, semg2, semg3, semg4,
                  sems0, sems1, sems2, sems3, sems4):
    c = lax.axis_index("c")
    s = lax.axis_index("s")
    wid = c * NS + s
    rows = (rows0, rows1, rows2, rows3, rows4)
    scl = (scl0, scl1, scl2, scl3, scl4)
    semg = (semg0, semg1, semg2, semg3, semg4)
    sems = (sems0, sems1, sems2, sems3, sems4)

    pltpu.sync_copy(src_hbm.at[wid], src_st)
    pltpu.sync_copy(dst_hbm.at[wid], dst_st)
    pltpu.sync_copy(w_hbm.at[wid], w_st)

    def zero_accum():
        def zrows(i, _):
            r = i // (HD // 16)
            j = i % (HD // 16)
            rows0[r, pl.ds(j * 16, 16)] = jnp.zeros((16,), jnp.float32)
            return 0
        lax.fori_loop(0, K * (HD // 16), zrows, 0, unroll=8)
        for i in range(RPT // K):
            pltpu.async_copy(rows0, s_sh.at[pl.ds(s * RPT + i * K, K)], semg0)
        for i in range(RPT // K):
            pltpu.make_async_copy(rows0, s_sh.at[pl.ds(s * RPT, K)],
                                  semg0).wait()

    GA = 2

    def run_pass(x_hbm, out_hbm):
        zero_accum()
        plsc.subcore_barrier()

        for b in range(GA):
            pltpu.async_copy(x_hbm.at[src_st.at[b]], rows[b], semg[b])

        def outer(t, _):
            for b in range(NB):
                ch = t * NB + b
                rb = rows[b]
                sb = scl[b]
                bn = (b + GA) % NB

                @pl.when(ch - NB >= 0)
                def _(sb=sb, b=b):
                    pltpu.make_async_copy(sb, s_sh.at[dst_st.at[0]],
                                          sems[b]).wait()

                @pl.when(ch + GA < CPW)
                def _(bn=bn, ch=ch):
                    pltpu.async_copy(x_hbm.at[src_st.at[ch + GA]], rows[bn],
                                     semg[bn])

                pltpu.make_async_copy(x_hbm.at[src_st.at[ch]], rb,
                                      semg[b]).wait()

                def grp(g, _, rb=rb, sb=sb, ch=ch):
                    wv = w_st[ch, pl.ds(g * 16, 16)]
                    for l in range(16):
                        wb = _lane_broadcast(wv, l)
                        e = g * 16 + l
                        for j in range(HD // 16):
                            sb[e, pl.ds(j * 16, 16)] = (
                                rb[e, pl.ds(j * 16, 16)] * wb)
                    return 0
                lax.fori_loop(0, K // 16, grp, 0)

                pltpu.async_copy(sb, s_sh.at[dst_st.at[ch]], sems[b],
                                 add=True)
            return 0
        lax.fori_loop(0, NT, outer, 0)

        for b in range(NB):
            pltpu.make_async_copy(scl[b], s_sh.at[dst_st.at[0]],
                                  sems[b]).wait()

        plsc.subcore_barrier()
        pltpu.sync_copy(s_sh.at[pl.ds(s * RPT, RPT)],
                        out_hbm.at[c, pl.ds(s * RPT, RPT)])
        plsc.subcore_barrier()

    run_pass(xlo_hbm, outlo_hbm)
    run_pass(xhi_hbm, outhi_hbm)


def _tc_gates_body(slo_ref, shi_ref, xlo_ref, xhi_ref, dis_ref,
                   wci, bci, wli, bli,
                   wcg, bcg, wlg, blg,
                   wco, bco, wlo, blo,
                   o_ref, hn_ref, cn_ref):
    dis = dis_ref[...]
    ylo = (slo_ref[0] + slo_ref[1] + xlo_ref[...]) * dis[:, None]
    yhi = (shi_ref[0] + shi_ref[1] + xhi_ref[...]) * dis[:, None]

    def z(wc, bc, wl, bl):
        wl_top = wl[:D, :]
        weff = jnp.dot(wc[...], wl_top, preferred_element_type=jnp.float32)
        beff = jnp.dot(bc[...], wl_top, preferred_element_type=jnp.float32) + bl[...]
        return (jnp.dot(ylo, weff[:HD, :], preferred_element_type=jnp.float32)
                + jnp.dot(yhi, weff[HD:, :], preferred_element_type=jnp.float32)
                + beff)

    gi = jax.nn.sigmoid(z(wci, bci, wli, bli))
    gg = jnp.tanh(z(wcg, bcg, wlg, blg))
    go = jax.nn.sigmoid(z(wco, bco, wlo, blo))
    cn = gi * gg
    o_ref[...] = go
    cn_ref[...] = cn
    hn_ref[...] = go * jnp.tanh(cn)


_ROWB = 2048
_GRID = NP // _ROWB


def _row_spec(cols=D):
    return pl.BlockSpec((_ROWB, cols), lambda i: (i, 0))


def _full_spec(shape):
    return pl.BlockSpec(shape, lambda i: (0,) * len(shape))


_tc_gates = pl.pallas_call(
    _tc_gates_body,
    grid=(_GRID,),
    in_specs=[
        pl.BlockSpec((NC, _ROWB, HD), lambda i: (0, i, 0)),
        pl.BlockSpec((NC, _ROWB, HD), lambda i: (0, i, 0)),
        _row_spec(HD),
        _row_spec(HD),
        pl.BlockSpec((_ROWB,), lambda i: (i,)),
    ] + [_full_spec((D, D)), _full_spec((1, D)),
         _full_spec((2 * D, D)), _full_spec((1, D))] * 3,
    out_specs=[_row_spec()] * 3,
    out_shape=[jax.ShapeDtypeStruct((N, D), jnp.float32)] * 3,
)


def kernel(X, edge_index, edge_weight,
           Wc_i, bc_i, Wl_i, bl_i,
           Wc_f, bc_f, Wl_f, bl_f,
           Wc_g, bc_g, Wl_g, bl_g,
           Wc_o, bc_o, Wl_o, bl_o):
    src3 = edge_index[0].reshape(NW, CPW, K)
    dst3 = edge_index[1].reshape(NW, CPW, K)
    w3 = edge_weight.reshape(NW, CPW, K)
    dst3d = edge_index[1].reshape(NS, CPT, K)
    w3d = edge_weight.reshape(NS, CPT, K)
    xp = jnp.pad(X, ((0, NP - N), (0, 0)))

    dis, xlo, xhi = _sc_degree(dst3d, w3d, xp)
    slo, shi = _sc_aggregate(src3, dst3, w3, xlo, xhi)

    r1 = lambda b: b.reshape(1, D)
    o, hn, cn = _tc_gates(
        slo, shi, xlo, xhi, dis,
        Wc_i, r1(bc_i), Wl_i, r1(bl_i),
        Wc_g, r1(bc_g), Wl_g, r1(bl_g),
        Wc_o, r1(bc_o), Wl_o, r1(bl_o),
    )
    return (o, hn, cn)
